# baseline (speedup 1.0000x reference)
import functools

import jax
import jax.numpy as jnp
import numpy as np
from jax.experimental import pallas as pl
from jax.experimental.pallas import tpu as pltpu

N = 8192
NSUB = 8
SUB = N // NSUB
K = int(SUB * 0.05)
ITERS = 5
JB = 256
NJB = N // JB

_MININT = np.int32(-2**31)


def _orderkey(x):
    f = jax.lax.bitcast_convert_type(x, jnp.int32)
    return jnp.where(f >= 0, f, jnp.bitwise_not(f ^ _MININT))


def _topk_mask(xa):
    key = _orderkey(xa)
    p = jnp.zeros((NSUB, 1), jnp.int32)
    for b in range(31, -1, -1):
        trial = p | np.int32(1 << b) if b < 31 else p | _MININT
        th = trial ^ _MININT
        cnt = jnp.sum((key >= th).astype(jnp.int32), axis=1, keepdims=True)
        p = jnp.where(cnt >= K, trial, p)
    vk = p ^ _MININT
    gt = key > vk
    eq = key == vk
    need = K - jnp.sum(gt.astype(jnp.int32), axis=1, keepdims=True)
    idx = jax.lax.broadcasted_iota(jnp.int32, (NSUB, SUB), 1)
    lo = jnp.zeros((NSUB, 1), jnp.int32)
    hi = jnp.full((NSUB, 1), SUB, jnp.int32)
    for _ in range(11):
        mid = (lo + hi) // 2
        c = jnp.sum((eq & (idx < mid)).astype(jnp.int32), axis=1, keepdims=True)
        ok = c >= need
        hi = jnp.where(ok, mid, hi)
        lo = jnp.where(ok, lo, mid + 1)
    sel = gt | (eq & (idx < hi))
    return sel.astype(jnp.float32)


def _body(h0_ref, noise_ref, w_ref, out_ref, acc_ref, h_ref):
    it = pl.program_id(0)
    jb = pl.program_id(1)

    @pl.when((it == 0) & (jb == 0))
    def _():
        h_ref[...] = h0_ref[...].astype(jnp.bfloat16).astype(jnp.float32)

    wf = w_ref[...].astype(jnp.bfloat16).astype(jnp.float32)
    r = jb // 4
    c0 = (jb % 4) * JB
    hseg = h_ref[pl.ds(r, 1), pl.ds(c0, JB)]
    p = wf * hseg
    folded = p[:, 0:128] + p[:, 128:256]

    @pl.when(jb == 0)
    def _():
        acc_ref[...] = folded

    @pl.when(jb != 0)
    def _():
        acc_ref[...] += folded

    @pl.when(jb == NJB - 1)
    def _():
        x64 = jnp.sum(acc_ref[...].reshape(64, 128, 128), axis=-1)
        xa = x64.reshape(NSUB, SUB)
        scale = (1e-10 + jnp.max(xa) - jnp.min(xa)) / 100000.0
        xn = xa + scale * noise_ref[0]
        mask = _topk_mask(xn)
        h_ref[...] = mask

        @pl.when(it == ITERS - 1)
        def _():
            out_ref[...] = mask


@jax.jit
def kernel(h_0, w):
    keys = []
    key = jax.random.key(42)
    for _ in range(ITERS):
        key, sub = jax.random.split(key)
        keys.append(sub)
    noise = jnp.stack(
        [jax.random.normal(k, (N,), jnp.float32) for k in keys]
    ).reshape(ITERS, NSUB, SUB)
    h0 = h_0.reshape(NSUB, SUB)

    out = pl.pallas_call(
        _body,
        grid=(ITERS, NJB),
        in_specs=[
            pl.BlockSpec((NSUB, SUB), lambda it, jb: (0, 0)),
            pl.BlockSpec((1, NSUB, SUB), lambda it, jb: (it, 0, 0)),
            pl.BlockSpec((N, JB), lambda it, jb: (0, jb)),
        ],
        out_specs=pl.BlockSpec((NSUB, SUB), lambda it, jb: (0, 0)),
        out_shape=jax.ShapeDtypeStruct((NSUB, SUB), jnp.float32),
        scratch_shapes=[
            pltpu.VMEM((N, 128), jnp.float32),
            pltpu.VMEM((NSUB, SUB), jnp.float32),
        ],
    )(h0, noise, w)
    return out.reshape(N)


# speedup vs baseline: 2.4657x; 2.4657x over previous
import jax
import jax.numpy as jnp
import numpy as np
from jax.experimental import pallas as pl
from jax.experimental.pallas import tpu as pltpu

N = 8192
NSUB = 8
SUB = N // NSUB
K = int(SUB * 0.05)
ITERS = 5
IB = 256
NIB = N // IB
CH = 1024
NCH = N // CH

_MININT = np.int32(-2**31)


def _orderkey(x):
    f = jax.lax.bitcast_convert_type(x, jnp.int32)
    return jnp.where(f >= 0, f, jnp.bitwise_not(f ^ _MININT))


def _topk_mask(xa):
    key = _orderkey(xa)
    p = jnp.zeros((NSUB, 1), jnp.int32)
    for b in range(31, -1, -1):
        trial = p | np.int32(1 << b) if b < 31 else p | _MININT
        th = trial ^ _MININT
        cnt = jnp.sum((key >= th).astype(jnp.int32), axis=1, keepdims=True)
        p = jnp.where(cnt >= K, trial, p)
    vk = p ^ _MININT
    gt = key > vk
    eq = key == vk
    need = K - jnp.sum(gt.astype(jnp.int32), axis=1, keepdims=True)
    idx = jax.lax.broadcasted_iota(jnp.int32, (NSUB, SUB), 1)
    lo = jnp.zeros((NSUB, 1), jnp.int32)
    hi = jnp.full((NSUB, 1), SUB, jnp.int32)
    for _ in range(11):
        mid = (lo + hi) // 2
        c = jnp.sum((eq & (idx < mid)).astype(jnp.int32), axis=1, keepdims=True)
        ok = c >= need
        hi = jnp.where(ok, mid, hi)
        lo = jnp.where(ok, lo, mid + 1)
    sel = gt | (eq & (idx < hi))
    return sel.astype(jnp.float32)


def _activation(xa, noise):
    scale = (1e-10 + jnp.max(xa) - jnp.min(xa)) / 100000.0
    return _topk_mask(xa + scale * noise)


def _stripe_matvec(get_chunk, h):
    acc = None
    for ch in range(NCH):
        wf = get_chunk(ch)
        p = wf * h[ch:ch + 1, :]
        f = p[:, 0:128]
        for q in range(1, CH // 128):
            f = f + p[:, 128 * q:128 * (q + 1)]
        acc = f if acc is None else acc + f
    return jnp.sum(acc.reshape(IB // 128, 128, 128), axis=-1)


def _body_a(h0_ref, w_ref, wbf_ref, x1_ref, hq_ref):
    ib = pl.program_id(0)

    @pl.when(ib == 0)
    def _():
        hq_ref[...] = h0_ref[...].astype(jnp.bfloat16).astype(jnp.float32)

    hq = hq_ref[...]

    def get_chunk(ch):
        wb = w_ref[:, CH * ch:CH * (ch + 1)].astype(jnp.bfloat16)
        wbf_ref[:, CH * ch:CH * (ch + 1)] = wb
        return wb.astype(jnp.float32)

    x1_ref[pl.ds(ib * (IB // 128), IB // 128), :] = _stripe_matvec(get_chunk, hq)


def _body_b(x1_ref, noise_ref, wbf_ref, out_ref, xs_ref, h_ref):
    it = pl.program_id(0)
    ib = pl.program_id(1)

    @pl.when((it == 0) & (ib == 0))
    def _():
        xa = x1_ref[...].reshape(NSUB, SUB)
        h_ref[...] = _activation(xa, noise_ref[0])

    h = h_ref[...]
    xs_ref[pl.ds(ib * (IB // 128), IB // 128), :] = _stripe_matvec(
        lambda c: wbf_ref[:, CH * c:CH * (c + 1)].astype(jnp.float32), h)

    @pl.when(ib == NIB - 1)
    def _():
        xa = xs_ref[...].reshape(NSUB, SUB)
        nz = noise_ref[pl.ds(it + 1, 1)][0]
        mask = _activation(xa, nz)
        h_ref[...] = mask

        @pl.when(it == ITERS - 2)
        def _():
            out_ref[...] = mask


@jax.jit
def kernel(h_0, w):
    keys = []
    key = jax.random.key(42)
    for _ in range(ITERS):
        key, sub = jax.random.split(key)
        keys.append(sub)
    noise = jnp.stack(
        [jax.random.normal(k, (N,), jnp.float32) for k in keys]
    ).reshape(ITERS, NSUB, SUB)
    h0 = h_0.reshape(NSUB, SUB)

    wbf, x1 = pl.pallas_call(
        _body_a,
        grid=(NIB,),
        in_specs=[
            pl.BlockSpec((NSUB, SUB), lambda ib: (0, 0)),
            pl.BlockSpec((IB, N), lambda ib: (ib, 0)),
        ],
        out_specs=[
            pl.BlockSpec((IB, N), lambda ib: (ib, 0)),
            pl.BlockSpec((N // 128, 128), lambda ib: (0, 0)),
        ],
        out_shape=[
            jax.ShapeDtypeStruct((N, N), jnp.bfloat16),
            jax.ShapeDtypeStruct((N // 128, 128), jnp.float32),
        ],
        scratch_shapes=[pltpu.VMEM((NSUB, SUB), jnp.float32)],
    )(h0, w)

    out = pl.pallas_call(
        _body_b,
        grid=(ITERS - 1, NIB),
        in_specs=[
            pl.BlockSpec((N // 128, 128), lambda it, ib: (0, 0)),
            pl.BlockSpec((ITERS, NSUB, SUB), lambda it, ib: (0, 0, 0)),
            pl.BlockSpec((IB, N), lambda it, ib: (ib, 0)),
        ],
        out_specs=pl.BlockSpec((NSUB, SUB), lambda it, ib: (0, 0)),
        out_shape=jax.ShapeDtypeStruct((NSUB, SUB), jnp.float32),
        scratch_shapes=[
            pltpu.VMEM((N // 128, 128), jnp.float32),
            pltpu.VMEM((NSUB, SUB), jnp.float32),
        ],
    )(x1, noise, wbf)
    return out.reshape(N)
